# Initial kernel scaffold; baseline (speedup 1.0000x reference)
import jax
import jax.numpy as jnp
from jax.experimental import pallas as pl


def kernel(confidences, regressions, anchors):
    raise NotImplementedError("write your pallas kernel here")


# speedup vs baseline: 10.8895x; 10.8895x over previous
import functools

import jax
import jax.numpy as jnp
from jax import lax
from jax.experimental import pallas as pl
from jax.experimental.pallas import tpu as pltpu

NUM_CLASSES = 90
TOP_K = 200
CONF_THRESH = 0.05
NMS_THRESH = 0.5
CROP_SIZE = 300.0
NEG = -1e9

B = 8
N = 20000
NBLK = 10
BN = N // NBLK
BIGI = 2**30


def _prep_kernel(conf_ref, reg_ref, anc_ref,
                 cur_ref, cls_ref, x1_ref, y1_ref, x2_ref, y2_ref):
    conf = conf_ref[0, 0]
    reg = reg_ref[0, 0]
    anc = anc_ref[0]

    mx = jnp.max(conf, axis=-1, keepdims=True)
    iot = lax.broadcasted_iota(jnp.int32, conf.shape, 1)
    cls = jnp.min(jnp.where(conf == mx, iot, BIGI), axis=-1)
    sc = mx[:, 0]
    clsf = cls.astype(jnp.float32)
    scm = jnp.where(sc > CONF_THRESH, sc, NEG)

    ya1, xa1, ya2, xa2 = anc[0], anc[1], anc[2], anc[3]
    yc_a = (ya1 + ya2) / 2.0
    xc_a = (xa1 + xa2) / 2.0
    ha = ya2 - ya1
    wa = xa2 - xa1
    w = jnp.exp(reg[3]) * wa
    h = jnp.exp(reg[2]) * ha
    yc = reg[0] * ha + yc_a
    xc = reg[1] * wa + xc_a
    x1 = jnp.clip(xc - w / 2.0, 0.0, CROP_SIZE) / CROP_SIZE
    y1 = jnp.clip(yc - h / 2.0, 0.0, CROP_SIZE) / CROP_SIZE
    x2 = jnp.clip(xc + w / 2.0, 0.0, CROP_SIZE) / CROP_SIZE
    y2 = jnp.clip(yc + h / 2.0, 0.0, CROP_SIZE) / CROP_SIZE

    off = clsf * 2.0
    cur_ref[0, 0, 0] = scm
    cls_ref[0, 0, 0] = clsf
    x1_ref[0, 0, 0] = x1 + off
    y1_ref[0, 0, 0] = y1 + off
    x2_ref[0, 0, 0] = x2 + off
    y2_ref[0, 0, 0] = y2 + off


def _nms_kernel(cur0_ref, cls_ref, x1_ref, y1_ref, x2_ref, y2_ref,
                ox1_ref, oy1_ref, ox2_ref, oy2_ref, osc_ref, ocl_ref):
    x1 = x1_ref[...]
    y1 = y1_ref[...]
    x2 = x2_ref[...]
    y2 = y2_ref[...]
    clsf = cls_ref[...]
    iota = lax.broadcasted_iota(jnp.int32, (B, N), 1)
    areas = jnp.clip(x2 - x1, 0.0, None) * jnp.clip(y2 - y1, 0.0, None)

    def body(t, cur):
        m = jnp.max(cur, axis=1, keepdims=True)
        valid = m > (NEG / 2.0)
        idxs = jnp.min(jnp.where(cur == m, iota, BIGI), axis=1, keepdims=True)
        onehot = iota == idxs

        def gather(plane):
            return jnp.sum(jnp.where(onehot, plane, 0.0), axis=1, keepdims=True)

        gx1 = gather(x1)
        gy1 = gather(y1)
        gx2 = gather(x2)
        gy2 = gather(y2)
        gcl = gather(clsf)

        xx1 = jnp.maximum(gx1, x1)
        yy1 = jnp.maximum(gy1, y1)
        xx2 = jnp.minimum(gx2, x2)
        yy2 = jnp.minimum(gy2, y2)
        inter = jnp.clip(xx2 - xx1, 0.0, None) * jnp.clip(yy2 - yy1, 0.0, None)
        area_i = jnp.clip(gx2 - gx1, 0.0, None) * jnp.clip(gy2 - gy1, 0.0, None)
        iou = inter / (area_i + areas - inter + 1e-8)
        supp = (iou > NMS_THRESH) | onehot
        new_cur = jnp.where(supp, NEG, cur)

        v = valid.astype(jnp.float32)
        off = gcl * 2.0
        ox1_ref[pl.ds(t, 1), :] = ((gx1 - off) * v).reshape(1, B)
        oy1_ref[pl.ds(t, 1), :] = ((gy1 - off) * v).reshape(1, B)
        ox2_ref[pl.ds(t, 1), :] = ((gx2 - off) * v).reshape(1, B)
        oy2_ref[pl.ds(t, 1), :] = ((gy2 - off) * v).reshape(1, B)
        osc_ref[pl.ds(t, 1), :] = (m * v).reshape(1, B)
        ocl_ref[pl.ds(t, 1), :] = jnp.where(valid, gcl, -1.0).reshape(1, B)
        return new_cur

    lax.fori_loop(0, TOP_K, body, cur0_ref[...])


@jax.jit
def kernel(confidences, regressions, anchors):
    conf4 = confidences.reshape(B, NBLK, BN, NUM_CLASSES)
    regs_t = regressions.reshape(B, NBLK, BN, 4).transpose(0, 1, 3, 2)
    anchors_t = anchors.reshape(NBLK, BN, 4).transpose(0, 2, 1)

    plane = jax.ShapeDtypeStruct((B, NBLK, 1, BN), jnp.float32)
    cur0, clsf, x1, y1, x2, y2 = pl.pallas_call(
        _prep_kernel,
        grid=(B, NBLK),
        in_specs=[
            pl.BlockSpec((1, 1, BN, NUM_CLASSES), lambda b, i: (b, i, 0, 0)),
            pl.BlockSpec((1, 1, 4, BN), lambda b, i: (b, i, 0, 0)),
            pl.BlockSpec((1, 4, BN), lambda b, i: (i, 0, 0)),
        ],
        out_specs=[pl.BlockSpec((1, 1, 1, BN), lambda b, i: (b, i, 0, 0))] * 6,
        out_shape=[plane] * 6,
        compiler_params=pltpu.CompilerParams(
            dimension_semantics=("parallel", "parallel")),
    )(conf4, regs_t, anchors_t)

    flat = lambda a: a.reshape(B, N)
    tk = jax.ShapeDtypeStruct((TOP_K, B), jnp.float32)
    ox1, oy1, ox2, oy2, osc, ocl = pl.pallas_call(
        _nms_kernel,
        out_shape=[tk] * 6,
    )(flat(cur0), flat(clsf), flat(x1), flat(y1), flat(x2), flat(y2))

    out = jnp.stack([ox1.T, oy1.T, ox2.T, oy2.T, osc.T], axis=-1)
    out_classes = ocl.T.astype(jnp.int32)
    return out, out_classes
